# Initial kernel scaffold; baseline (speedup 1.0000x reference)
import jax
import jax.numpy as jnp
from jax.experimental import pallas as pl


def kernel(x, edge_index, batch, params):
    raise NotImplementedError("write your pallas kernel here")


# speedup vs baseline: 2.9726x; 2.9726x over previous
import functools

import jax
import jax.numpy as jnp
from jax import lax
from jax.experimental import pallas as pl
from jax.experimental.pallas import tpu as pltpu
from jax.experimental.pallas import tpu_sc as plsc

N = 10000
E = 320000
D0 = 128
H = 32
G = 256
EPS = 1e-5

NC = 2
NS = 16
NW = NC * NS
CHUNK = 128
NCH = 80
E_PAD = NW * NCH * CHUNK
N_PAD = 10112
RPT = N_PAD // NS
INIT_CHUNKS = (128, 128, 128, 128, 120)
DUMMY = N

_f32 = jnp.float32
_i32 = jnp.int32


@functools.cache
def _get_segsum(d):
    mesh = plsc.VectorSubcoreMesh(core_axis_name="c", subcore_axis_name="s")

    @functools.partial(
        pl.kernel,
        out_type=jax.ShapeDtypeStruct((NC, N_PAD, d), _f32),
        mesh=mesh,
        compiler_params=pltpu.CompilerParams(use_tc_tiling_on_sc=False),
        scratch_types=[
            pltpu.VMEM((NCH, CHUNK), _i32),
            pltpu.VMEM((NCH, CHUNK), _i32),
            pltpu.VMEM((NCH * CHUNK + 16,), _f32),
            pltpu.VMEM((CHUNK, d), _f32),
            pltpu.VMEM((CHUNK, d), _f32),
            pltpu.VMEM_SHARED((N_PAD, d), _f32),
            pltpu.SemaphoreType.DMA,
            pltpu.SemaphoreType.DMA,
        ],
    )
    def segsum(p_hbm, src_hbm, dst2_hbm, keep_hbm, zeros_hbm, out_hbm,
               src_v, dst_v, keep_v, buf_a, buf_b, acc, sem_a, sem_b):
        c = lax.axis_index("c")
        s = lax.axis_index("s")
        w = c * NS + s
        r0 = s * RPT
        pltpu.sync_copy(zeros_hbm, buf_a)
        off = 0
        for sz in INIT_CHUNKS:
            pltpu.sync_copy(buf_a.at[pl.ds(0, sz)], acc.at[pl.ds(r0 + off, sz)])
            off += sz
        plsc.subcore_barrier()
        pltpu.sync_copy(src_hbm.at[pl.ds(w * NCH, NCH)], src_v)
        pltpu.sync_copy(dst2_hbm.at[pl.ds(w * NCH, NCH)], dst_v)
        pltpu.sync_copy(keep_hbm.at[pl.ds(w * NCH * CHUNK, NCH * CHUNK + 16)],
                        keep_v)
        nf = d // 16
        zidx = jnp.zeros((16,), _i32)

        def reduce_chunk(j, buf):
            def edge(e, accs):
                kv = keep_v[pl.ds(j * CHUNK + e, 16)]
                k = lax.gather(
                    kv, zidx[:, None],
                    lax.GatherDimensionNumbers(
                        offset_dims=(), collapsed_slice_dims=(0,),
                        start_index_map=(0,)),
                    slice_sizes=(1,),
                    mode=lax.GatherScatterMode.PROMISE_IN_BOUNDS)
                new = []
                for f in range(nf):
                    row = buf[e, pl.ds(f * 16, 16)]
                    a = accs[f] * k + row
                    buf[e, pl.ds(f * 16, 16)] = a
                    new.append(a)
                return tuple(new)

            lax.fori_loop(0, CHUNK, edge,
                          tuple(jnp.zeros((16,), _f32) for _ in range(nf)))

        def body(i, carry):
            ja = 2 * i
            jb = 2 * i + 1
            ga = pltpu.async_copy(p_hbm.at[src_v.at[ja]], buf_a, sem_a)
            gb = pltpu.async_copy(p_hbm.at[src_v.at[jb]], buf_b, sem_b)
            ga.wait()
            reduce_chunk(ja, buf_a)
            pltpu.sync_copy(buf_a, acc.at[dst_v.at[ja]], add=True)
            gb.wait()
            reduce_chunk(jb, buf_b)
            pltpu.sync_copy(buf_b, acc.at[dst_v.at[jb]], add=True)
            return carry

        lax.fori_loop(0, NCH // 2, body, 0)
        plsc.subcore_barrier()
        off = 0
        for sz in INIT_CHUNKS:
            pltpu.sync_copy(acc.at[pl.ds(r0 + off, sz)], buf_a.at[pl.ds(0, sz)])
            pltpu.sync_copy(buf_a.at[pl.ds(0, sz)], out_hbm.at[c, pl.ds(r0 + off, sz)])
            off += sz

    return segsum


def _mlp_bn(z, w1_ref, b1_ref, w2_ref, b2_ref, gam_ref, bet_ref):
    z = jnp.maximum(jnp.dot(z, w1_ref[...], preferred_element_type=_f32)
                    + b1_ref[...], 0.0)
    z = jnp.maximum(jnp.dot(z, w2_ref[...], preferred_element_type=_f32)
                    + b2_ref[...], 0.0)
    m = jnp.mean(z, axis=0, keepdims=True)
    v = jnp.mean((z - m) * (z - m), axis=0, keepdims=True)
    return (z - m) / jnp.sqrt(v + EPS) * gam_ref[...] + bet_ref[...]


def _layer_body(h_ref, q_ref, w1_ref, b1_ref, w2_ref, b2_ref, gam_ref,
                bet_ref, o_ref):
    z = h_ref[...] + (q_ref[0, :N, :] + q_ref[1, :N, :])
    o_ref[...] = _mlp_bn(z, w1_ref, b1_ref, w2_ref, b2_ref, gam_ref, bet_ref)


def _layer0_body(h_ref, qa_ref, qb_ref, w1_ref, b1_ref, w2_ref, b2_ref,
                 gam_ref, bet_ref, o_ref):
    agg = jnp.concatenate(
        [qa_ref[0, :N, :] + qa_ref[1, :N, :],
         qb_ref[0, :N, :] + qb_ref[1, :N, :]], axis=1)
    z = h_ref[...] + agg
    o_ref[...] = _mlp_bn(z, w1_ref, b1_ref, w2_ref, b2_ref, gam_ref, bet_ref)


def _layer(h, qs, layer):
    body = _layer_body if len(qs) == 1 else _layer0_body
    return pl.pallas_call(
        body, out_shape=jax.ShapeDtypeStruct((N, H), _f32))(
            h, *qs, layer["W1"], layer["b1"].reshape(1, H), layer["W2"],
            layer["b2"].reshape(1, H), layer["gamma"].reshape(1, H),
            layer["beta"].reshape(1, H))


def _final_body(h_ref, q_ref, w1_ref, b1_ref, w2_ref, b2_ref, gam_ref,
                bet_ref, batch_ref, wfc_ref, bfc_ref, o_ref):
    z = h_ref[...] + (q_ref[0, :N, :] + q_ref[1, :N, :])
    h = _mlp_bn(z, w1_ref, b1_ref, w2_ref, b2_ref, gam_ref, bet_ref)
    h = jnp.concatenate([h, jnp.zeros((N_PAD - N, H), _f32)], axis=0)
    onehot = (lax.broadcasted_iota(_i32, (G, N_PAD), 0)
              == batch_ref[...]).astype(_f32)
    pooled = jnp.dot(onehot, h, preferred_element_type=_f32,
                     precision=lax.Precision.HIGHEST)
    o_ref[...] = jnp.maximum(
        jnp.dot(pooled, wfc_ref[...], preferred_element_type=_f32)
        + bfc_ref[...], 0.0)


def _final(h, q, layer, batch2, wfc, bfc):
    return pl.pallas_call(
        _final_body, out_shape=jax.ShapeDtypeStruct((G, wfc.shape[1]), _f32))(
            h, q, layer["W1"], layer["b1"].reshape(1, H), layer["W2"],
            layer["b2"].reshape(1, H), layer["gamma"].reshape(1, H),
            layer["beta"].reshape(1, H), batch2, wfc, bfc.reshape(1, -1))


def kernel(x, edge_index, batch, params):
    layers = params["layers"]
    perm = jnp.argsort(edge_index[1], stable=True)
    pad = E_PAD - E
    src_p = jnp.concatenate(
        [edge_index[0][perm], jnp.zeros((pad,), _i32)]).reshape(NW * NCH, CHUNK)
    dst_s = jnp.concatenate(
        [edge_index[1][perm], DUMMY + (jnp.arange(pad, dtype=_i32) % 16)])
    epos = jnp.arange(E_PAD, dtype=_i32)
    same_prev = jnp.concatenate(
        [jnp.zeros((1,), jnp.bool_), dst_s[1:] == dst_s[:-1]])
    keep = (same_prev & (epos % CHUNK != 0)).astype(_f32)
    run_end = jnp.concatenate(
        [dst_s[:-1] != dst_s[1:], jnp.ones((1,), jnp.bool_)]) | (
            epos % CHUNK == CHUNK - 1)
    dst2 = jnp.where(run_end, dst_s, DUMMY + (epos % 96))
    keep_p = jnp.concatenate([keep, jnp.zeros((16,), _f32)])
    dst_p = dst2.reshape(NW * NCH, CHUNK)
    batch2 = jnp.concatenate(
        [batch, jnp.full((N_PAD - N,), -1, _i32)]).reshape(1, N_PAD)
    zeros0 = jnp.zeros((CHUNK, D0 // 2), _f32)
    zerosh = jnp.zeros((CHUNK, H), _f32)

    qa = _get_segsum(D0 // 2)(x[:, :D0 // 2], src_p, dst_p, keep_p, zeros0)
    qb = _get_segsum(D0 // 2)(x[:, D0 // 2:], src_p, dst_p, keep_p, zeros0)
    h = _layer(x, (qa, qb), layers[0])
    for l in range(1, 4):
        q = _get_segsum(H)(h, src_p, dst_p, keep_p, zerosh)
        h = _layer(h, (q,), layers[l])
    q = _get_segsum(H)(h, src_p, dst_p, keep_p, zerosh)
    return _final(h, q, layers[4], batch2, params["Wfc"], params["bfc"])
